# baseline (device time: 20122 ns/iter reference)
import jax
import jax.numpy as jnp
from jax import lax
from jax.experimental import pallas as pl
from jax.experimental.pallas import tpu as pltpu

N_DEV = 32


def kernel(x):
    m_per, n = x.shape

    def body(
        x_ref,
        out_ref,
        send_buf,
        recv_bufs,
        send_sems,
        recv_sems,
        ack_sems,
        entry_sems,
    ):
        my = lax.axis_index("i")
        ones = jnp.ones((1, n), jnp.float32)

        barrier_sem = pltpu.get_barrier_semaphore()
        for nbr in ((my + 1) % N_DEV, (my + N_DEV - 1) % N_DEV):
            pl.semaphore_signal(
                barrier_sem,
                inc=1,
                device_id=(nbr,),
                device_id_type=pl.DeviceIdType.MESH,
            )
        pl.semaphore_wait(barrier_sem, 2)

        for k in range(N_DEV - 1):

            @pl.when(my >= k + 1)
            def _(k=k):
                pl.semaphore_signal(
                    entry_sems.at[k],
                    inc=1,
                    device_id=(my - 1 - k,),
                    device_id_type=pl.DeviceIdType.MESH,
                )

        t = x_ref[:, :]
        h = m_per
        while h > 1:
            h //= 2
            t = t[:h, :] * t[h : 2 * h, :]
        send_buf[:, :] = t

        def descriptor(k):
            return pltpu.make_async_remote_copy(
                src_ref=send_buf,
                dst_ref=recv_bufs.at[k],
                send_sem=send_sems.at[k],
                recv_sem=recv_sems.at[k],
                device_id=(my + 1 + k,),
                device_id_type=pl.DeviceIdType.MESH,
            )

        for k in range(N_DEV - 1):

            @pl.when(my + 1 + k < N_DEV)
            def _(k=k):
                pl.semaphore_wait(entry_sems.at[k], 1)
                descriptor(k).start()

        a = x_ref[:, :]
        kk = 1
        while kk < m_per:
            shifted = jnp.concatenate(
                [jnp.ones((kk, n), jnp.float32), a[: m_per - kk, :]], axis=0
            )
            a = a * shifted
            kk *= 2

        e = ones
        for k in range(N_DEV - 1):

            @pl.when(my >= k + 1)
            def _(k=k):
                recv = pltpu.make_async_remote_copy(
                    src_ref=send_buf,
                    dst_ref=recv_bufs.at[k],
                    send_sem=send_sems.at[k],
                    recv_sem=recv_sems.at[k],
                    device_id=(my - 1 - k,),
                    device_id_type=pl.DeviceIdType.MESH,
                )
                recv.wait_recv()
                pl.semaphore_signal(
                    ack_sems.at[k],
                    inc=1,
                    device_id=(my - 1 - k,),
                    device_id_type=pl.DeviceIdType.MESH,
                )

            e = e * jnp.where(my >= k + 1, recv_bufs[k, :, :], ones)

        out_ref[:, :] = a * e

        for k in range(N_DEV - 1):

            @pl.when(my + 1 + k < N_DEV)
            def _(k=k):
                descriptor(k).wait_send()
                pl.semaphore_wait(ack_sems.at[k], 1)

    return pl.pallas_call(
        body,
        out_shape=jax.ShapeDtypeStruct((m_per, n), jnp.float32),
        in_specs=[pl.BlockSpec(memory_space=pltpu.VMEM)],
        out_specs=pl.BlockSpec(memory_space=pltpu.VMEM),
        scratch_shapes=[
            pltpu.VMEM((1, n), jnp.float32),
            pltpu.VMEM((N_DEV - 1, 1, n), jnp.float32),
            pltpu.SemaphoreType.DMA((N_DEV - 1,)),
            pltpu.SemaphoreType.DMA((N_DEV - 1,)),
            pltpu.SemaphoreType.REGULAR((N_DEV - 1,)),
            pltpu.SemaphoreType.REGULAR((N_DEV - 1,)),
        ],
        compiler_params=pltpu.CompilerParams(collective_id=0),
    )(x)


# device time: 17036 ns/iter; 1.1811x vs baseline; 1.1811x over previous
import jax
import jax.numpy as jnp
from jax import lax
from jax.experimental import pallas as pl
from jax.experimental.pallas import tpu as pltpu

N_DEV = 32

ROUNDS = ((1, 2, 3), (4, 8, 12), (16,))
SLOTS = [(r, d) for r, dists in enumerate(ROUNDS) for d in dists]
N_SLOTS = len(SLOTS)


def kernel(x):
    m_per, n = x.shape

    def body(
        x_ref,
        out_ref,
        send_bufs,
        recv_bufs,
        send_sems,
        recv_sems,
        ack_sems,
        entry_sems,
    ):
        my = lax.axis_index("i")
        ones = jnp.ones((1, n), jnp.float32)

        barrier_sem = pltpu.get_barrier_semaphore()
        for nbr in ((my + 1) % N_DEV, (my + N_DEV - 1) % N_DEV):
            pl.semaphore_signal(
                barrier_sem,
                inc=1,
                device_id=(nbr,),
                device_id_type=pl.DeviceIdType.MESH,
            )
        pl.semaphore_wait(barrier_sem, 2)

        for slot, (_, d) in enumerate(SLOTS):

            @pl.when(my >= d)
            def _(slot=slot, d=d):
                pl.semaphore_signal(
                    entry_sems.at[slot],
                    inc=1,
                    device_id=(my - d,),
                    device_id_type=pl.DeviceIdType.MESH,
                )

        t = x_ref[:, :]
        h = m_per
        while h > 1:
            h //= 2
            t = t[:h, :] * t[h : 2 * h, :]

        def send_descriptor(slot, rnd, d):
            return pltpu.make_async_remote_copy(
                src_ref=send_bufs.at[rnd],
                dst_ref=recv_bufs.at[slot],
                send_sem=send_sems.at[slot],
                recv_sem=recv_sems.at[slot],
                device_id=(my + d,),
                device_id_type=pl.DeviceIdType.MESH,
            )

        r = t
        e = ones
        a = None

        slot = 0
        for rnd, dists in enumerate(ROUNDS):
            send_bufs[rnd, :, :] = r
            for d in dists:

                @pl.when(my + d < N_DEV)
                def _(slot=slot, rnd=rnd, d=d):
                    pl.semaphore_wait(entry_sems.at[slot], 1)
                    send_descriptor(slot, rnd, d).start()

                slot += 1

            if rnd == 0:
                a = x_ref[:, :]
                k = 1
                while k < m_per:
                    shifted = jnp.concatenate(
                        [jnp.ones((k, n), jnp.float32), a[: m_per - k, :]],
                        axis=0,
                    )
                    a = a * shifted
                    k *= 2

            slot -= len(dists)
            for d in dists:

                @pl.when(my >= d)
                def _(slot=slot, d=d):
                    recv = pltpu.make_async_remote_copy(
                        src_ref=send_bufs.at[0],
                        dst_ref=recv_bufs.at[slot],
                        send_sem=send_sems.at[slot],
                        recv_sem=recv_sems.at[slot],
                        device_id=(my - d,),
                        device_id_type=pl.DeviceIdType.MESH,
                    )
                    recv.wait_recv()
                    pl.semaphore_signal(
                        ack_sems.at[slot],
                        inc=1,
                        device_id=(my - d,),
                        device_id_type=pl.DeviceIdType.MESH,
                    )

                v = jnp.where(my >= d, recv_bufs[slot, :, :], ones)
                e = e * v
                r = r * v
                slot += 1

        out_ref[:, :] = a * e

        for slot, (rnd, d) in enumerate(SLOTS):

            @pl.when(my + d < N_DEV)
            def _(slot=slot, rnd=rnd, d=d):
                send_descriptor(slot, rnd, d).wait_send()
                pl.semaphore_wait(ack_sems.at[slot], 1)

    return pl.pallas_call(
        body,
        out_shape=jax.ShapeDtypeStruct((m_per, n), jnp.float32),
        in_specs=[pl.BlockSpec(memory_space=pltpu.VMEM)],
        out_specs=pl.BlockSpec(memory_space=pltpu.VMEM),
        scratch_shapes=[
            pltpu.VMEM((len(ROUNDS), 1, n), jnp.float32),
            pltpu.VMEM((N_SLOTS, 1, n), jnp.float32),
            pltpu.SemaphoreType.DMA((N_SLOTS,)),
            pltpu.SemaphoreType.DMA((N_SLOTS,)),
            pltpu.SemaphoreType.REGULAR((N_SLOTS,)),
            pltpu.SemaphoreType.REGULAR((N_SLOTS,)),
        ],
        compiler_params=pltpu.CompilerParams(collective_id=0),
    )(x)


# device time: 16063 ns/iter; 1.2527x vs baseline; 1.0606x over previous
import jax
import jax.numpy as jnp
from jax import lax
from jax.experimental import pallas as pl
from jax.experimental.pallas import tpu as pltpu

N_DEV = 32
N_STEPS = 5


def kernel(x):
    m_per, n = x.shape

    def body(
        x_ref,
        out_ref,
        send_bufs,
        recv_bufs,
        send_sems,
        recv_sems,
        ack_sems,
        entry_sems,
    ):
        my = lax.axis_index("i")
        ones = jnp.ones((1, n), jnp.float32)

        barrier_sem = pltpu.get_barrier_semaphore()
        for nbr in ((my + 1) % N_DEV, (my + N_DEV - 1) % N_DEV):
            pl.semaphore_signal(
                barrier_sem,
                inc=1,
                device_id=(nbr,),
                device_id_type=pl.DeviceIdType.MESH,
            )
        pl.semaphore_wait(barrier_sem, 2)

        for s in range(N_STEPS):
            d = 1 << s

            @pl.when(my >= d)
            def _(s=s, d=d):
                pl.semaphore_signal(
                    entry_sems.at[s],
                    inc=1,
                    device_id=(my - d,),
                    device_id_type=pl.DeviceIdType.MESH,
                )

        t = x_ref[:, :]
        h = m_per
        while h > 1:
            h //= 2
            t = t[:h, :] * t[h : 2 * h, :]

        r = t
        e = ones
        a = None

        for s in range(N_STEPS):
            d = 1 << s
            send_bufs[s, :, :] = r

            @pl.when(my + d < N_DEV)
            def _(s=s, d=d):
                pl.semaphore_wait(entry_sems.at[s], 1)
                send = pltpu.make_async_remote_copy(
                    src_ref=send_bufs.at[s],
                    dst_ref=recv_bufs.at[s],
                    send_sem=send_sems.at[s],
                    recv_sem=recv_sems.at[s],
                    device_id=(my + d,),
                    device_id_type=pl.DeviceIdType.MESH,
                )
                send.start()

            if s == 0:
                a = x_ref[:, :]
                k = 1
                while k < m_per:
                    shifted = jnp.concatenate(
                        [jnp.ones((k, n), jnp.float32), a[: m_per - k, :]],
                        axis=0,
                    )
                    a = a * shifted
                    k *= 2

            @pl.when(my >= d)
            def _(s=s, d=d):
                recv = pltpu.make_async_remote_copy(
                    src_ref=send_bufs.at[s],
                    dst_ref=recv_bufs.at[s],
                    send_sem=send_sems.at[s],
                    recv_sem=recv_sems.at[s],
                    device_id=(my - d,),
                    device_id_type=pl.DeviceIdType.MESH,
                )
                recv.wait_recv()
                pl.semaphore_signal(
                    ack_sems.at[s],
                    inc=1,
                    device_id=(my - d,),
                    device_id_type=pl.DeviceIdType.MESH,
                )

            v = jnp.where(my >= d, recv_bufs[s, :, :], ones)
            e = e * v
            r = r * v

        out_ref[:, :] = a * e

        for s in range(N_STEPS):
            d = 1 << s

            @pl.when(my + d < N_DEV)
            def _(s=s, d=d):
                send = pltpu.make_async_remote_copy(
                    src_ref=send_bufs.at[s],
                    dst_ref=recv_bufs.at[s],
                    send_sem=send_sems.at[s],
                    recv_sem=recv_sems.at[s],
                    device_id=(my + d,),
                    device_id_type=pl.DeviceIdType.MESH,
                )
                send.wait_send()
                pl.semaphore_wait(ack_sems.at[s], 1)

    return pl.pallas_call(
        body,
        out_shape=jax.ShapeDtypeStruct((m_per, n), jnp.float32),
        in_specs=[pl.BlockSpec(memory_space=pltpu.VMEM)],
        out_specs=pl.BlockSpec(memory_space=pltpu.VMEM),
        scratch_shapes=[
            pltpu.VMEM((N_STEPS, 1, n), jnp.float32),
            pltpu.VMEM((N_STEPS, 1, n), jnp.float32),
            pltpu.SemaphoreType.DMA((N_STEPS,)),
            pltpu.SemaphoreType.DMA((N_STEPS,)),
            pltpu.SemaphoreType.REGULAR((N_STEPS,)),
            pltpu.SemaphoreType.REGULAR((N_STEPS,)),
        ],
        compiler_params=pltpu.CompilerParams(collective_id=0),
    )(x)
